# baseline (speedup 1.0000x reference)
import functools

import jax
import jax.numpy as jnp
from jax import lax
from jax.experimental import pallas as pl
from jax.experimental.pallas import tpu as pltpu

_BMA = 400
_BMO = 200
_BMP = 2000


def _p_body(f_ref, wb_ref, wm_ref, wl_ref, p_ref):
    wcat = jnp.concatenate([wm_ref[...], wl_ref[...]], axis=1)
    wc = jnp.dot(wb_ref[...], wcat, preferred_element_type=jnp.float32)
    p_ref[...] = jnp.dot(f_ref[...], wc, preferred_element_type=jnp.float32)


def _body(adj_ref, p_ref, noise_ref, o_ref, s_ref, *, nc, nba, d_emb):
    i = pl.program_id(0)
    d2 = 2 * d_emb
    g0, z0 = nc, nc + nba
    o0 = nc + 2 * nba

    @pl.when(i < nc)
    def _phase_copy():
        s_ref[pl.ds(i * _BMP, _BMP), :d2] = p_ref[...]

    @pl.when((i >= g0) & (i < z0))
    def _phase_g():
        r = (i - g0) * _BMA
        s_ref[pl.ds(r, _BMA), d2:2 * d2] = jnp.dot(
            adj_ref[...], s_ref[:, :d2],
            preferred_element_type=jnp.float32)

    @pl.when((i >= z0) & (i < o0))
    def _phase_z():
        r = (i - z0) * _BMA
        m = jnp.maximum(jnp.dot(adj_ref[...], s_ref[:, d2:2 * d2],
                                preferred_element_type=jnp.float32), 0.0)
        mean = m[:, :d_emb]
        logstd = m[:, d_emb:]
        s_ref[pl.ds(r, _BMA), 2 * d2:2 * d2 + d_emb] = (
            noise_ref[...] * jnp.exp(logstd) + mean)

    @pl.when(i >= o0)
    def _phase_out():
        r = (i - o0) * _BMO
        zi = s_ref[pl.ds(r, _BMO), 2 * d2:2 * d2 + d_emb]
        zall = s_ref[:, 2 * d2:2 * d2 + d_emb]
        o_ref[...] = lax.dot_general(
            zi, zall, (((1,), (1,)), ((), ())),
            preferred_element_type=jnp.float32)


def kernel(adj, features, W_base, W_mean, W_logstd, noise):
    n, d_in = features.shape
    d_hid = W_base.shape[1]
    d_emb = W_mean.shape[1]
    d2 = 2 * d_emb
    nc = n // _BMP
    nba = n // _BMA
    nbo = n // _BMO
    g0, z0, o0 = nc, nc + nba, nc + 2 * nba

    p = pl.pallas_call(
        _p_body,
        out_shape=jax.ShapeDtypeStruct((n, d2), jnp.float32),
    )(features, W_base, W_mean, W_logstd)

    def adj_map(i):
        return (jnp.where(i < z0, jnp.clip(i - g0, 0, nba - 1),
                          jnp.where(i < o0, i - z0, nba - 1)), 0)

    def p_map(i):
        return (jnp.clip(i, 0, nc - 1), 0)

    def noise_map(i):
        return (jnp.clip(i - z0, 0, nba - 1), 0)

    def out_map(i):
        return (jnp.where(i >= o0, i - o0, 0), 0)

    body = functools.partial(_body, nc=nc, nba=nba, d_emb=d_emb)

    out = pl.pallas_call(
        body,
        grid=(nc + 2 * nba + nbo,),
        in_specs=[
            pl.BlockSpec((_BMA, n), adj_map),
            pl.BlockSpec((_BMP, d2), p_map),
            pl.BlockSpec((_BMA, d_emb), noise_map),
        ],
        out_specs=pl.BlockSpec((_BMO, n), out_map),
        out_shape=jax.ShapeDtypeStruct((n, n), jnp.float32),
        scratch_shapes=[
            pltpu.VMEM((n, 2 * d2 + d_emb), jnp.float32),
        ],
    )(adj, p, noise)

    return out


# speedup vs baseline: 1.0011x; 1.0011x over previous
import functools

import jax
import jax.numpy as jnp
from jax import lax
from jax.experimental import pallas as pl
from jax.experimental.pallas import tpu as pltpu

_BMA = 400
_BMO = 200
_BMP = 2000


def _p_body(f_ref, wb_ref, wm_ref, wl_ref, p_ref):
    wcat = jnp.concatenate([wm_ref[...], wl_ref[...]], axis=1)
    wc = jnp.dot(wb_ref[...], wcat, preferred_element_type=jnp.float32)
    p_ref[...] = jnp.dot(f_ref[...], wc, preferred_element_type=jnp.float32)


def _body(adj_ref, p_ref, noise_ref, o_ref, s_ref, *, nc, nba, d_emb):
    i = pl.program_id(0)
    d2 = 2 * d_emb
    g0, z0 = nc, nc + nba
    o0 = nc + 2 * nba

    @pl.when(i < nc)
    def _phase_copy():
        s_ref[pl.ds(i * _BMP, _BMP), :d2] = p_ref[...]

    @pl.when((i >= g0) & (i < z0))
    def _phase_g():
        r = (i - g0) * _BMA
        s_ref[pl.ds(r, _BMA), d2:2 * d2] = jnp.dot(
            adj_ref[...], s_ref[:, :d2],
            precision=lax.Precision.DEFAULT,
            preferred_element_type=jnp.float32)

    @pl.when((i >= z0) & (i < o0))
    def _phase_z():
        r = (i - z0) * _BMA
        m = jnp.maximum(jnp.dot(adj_ref[...], s_ref[:, d2:2 * d2],
                                precision=lax.Precision.DEFAULT,
                                preferred_element_type=jnp.float32), 0.0)
        mean = m[:, :d_emb]
        logstd = m[:, d_emb:]
        s_ref[pl.ds(r, _BMA), 2 * d2:2 * d2 + d_emb] = (
            noise_ref[...] * jnp.exp(logstd) + mean)

    @pl.when(i >= o0)
    def _phase_out():
        r = (i - o0) * _BMO
        zi = s_ref[pl.ds(r, _BMO), 2 * d2:2 * d2 + d_emb]
        zall = s_ref[:, 2 * d2:2 * d2 + d_emb]
        o_ref[...] = lax.dot_general(
            zi, zall, (((1,), (1,)), ((), ())),
            precision=lax.Precision.DEFAULT,
            preferred_element_type=jnp.float32)


def kernel(adj, features, W_base, W_mean, W_logstd, noise):
    n, d_in = features.shape
    d_hid = W_base.shape[1]
    d_emb = W_mean.shape[1]
    d2 = 2 * d_emb
    nc = n // _BMP
    nba = n // _BMA
    nbo = n // _BMO
    g0, z0, o0 = nc, nc + nba, nc + 2 * nba

    p = pl.pallas_call(
        _p_body,
        out_shape=jax.ShapeDtypeStruct((n, d2), jnp.float32),
    )(features, W_base, W_mean, W_logstd)

    def adj_map(i):
        return (jnp.where(i < z0, jnp.clip(i - g0, 0, nba - 1),
                          jnp.where(i < o0, i - z0, nba - 1)), 0)

    def p_map(i):
        return (jnp.clip(i, 0, nc - 1), 0)

    def noise_map(i):
        return (jnp.clip(i - z0, 0, nba - 1), 0)

    def out_map(i):
        return (jnp.where(i >= o0, i - o0, 0), 0)

    body = functools.partial(_body, nc=nc, nba=nba, d_emb=d_emb)

    out = pl.pallas_call(
        body,
        grid=(nc + 2 * nba + nbo,),
        in_specs=[
            pl.BlockSpec((_BMA, n), adj_map),
            pl.BlockSpec((_BMP, d2), p_map),
            pl.BlockSpec((_BMA, d_emb), noise_map),
        ],
        out_specs=pl.BlockSpec((_BMO, n), out_map),
        out_shape=jax.ShapeDtypeStruct((n, n), jnp.float32),
        scratch_shapes=[
            pltpu.VMEM((n, 2 * d2 + d_emb), jnp.float32),
        ],
    )(adj, p, noise)

    return out


# speedup vs baseline: 1.0015x; 1.0003x over previous
import functools

import jax
import jax.numpy as jnp
from jax import lax
from jax.experimental import pallas as pl
from jax.experimental.pallas import tpu as pltpu

_BMA = 400
_BMO = 200
_BMP = 2000


def _p_body(f_ref, wb_ref, wm_ref, wl_ref, p_ref):
    wcat = jnp.concatenate([wm_ref[...], wl_ref[...]], axis=1)
    wc = jnp.dot(wb_ref[...], wcat, preferred_element_type=jnp.float32)
    p_ref[...] = jnp.dot(f_ref[...], wc, preferred_element_type=jnp.float32)


def _body(adj_ref, p_ref, noise_ref, o_ref, sp_ref, sg_ref, sz_ref,
          *, nc, nba, d_emb):
    i = pl.program_id(0)
    d2 = 2 * d_emb
    g0, z0 = nc, nc + nba
    o0 = nc + 2 * nba

    @pl.when(i < nc)
    def _phase_copy():
        sp_ref[pl.ds(i * _BMP, _BMP), :] = p_ref[...]

    @pl.when((i >= g0) & (i < z0))
    def _phase_g():
        r = (i - g0) * _BMA
        sg_ref[pl.ds(r, _BMA), :] = jnp.dot(
            adj_ref[...], sp_ref[...],
            precision=lax.Precision.DEFAULT,
            preferred_element_type=jnp.float32)

    @pl.when((i >= z0) & (i < o0))
    def _phase_z():
        r = (i - z0) * _BMA
        m = jnp.maximum(jnp.dot(adj_ref[...], sg_ref[...],
                                precision=lax.Precision.DEFAULT,
                                preferred_element_type=jnp.float32), 0.0)
        mean = m[:, :d_emb]
        logstd = m[:, d_emb:]
        sz_ref[pl.ds(r, _BMA), :] = (
            noise_ref[...] * jnp.exp(logstd) + mean)

    @pl.when(i >= o0)
    def _phase_out():
        r = (i - o0) * _BMO
        zi = sz_ref[pl.ds(r, _BMO), :]
        o_ref[...] = lax.dot_general(
            zi, sz_ref[...], (((1,), (1,)), ((), ())),
            precision=lax.Precision.DEFAULT,
            preferred_element_type=jnp.float32)


def kernel(adj, features, W_base, W_mean, W_logstd, noise):
    n, d_in = features.shape
    d_hid = W_base.shape[1]
    d_emb = W_mean.shape[1]
    d2 = 2 * d_emb
    nc = n // _BMP
    nba = n // _BMA
    nbo = n // _BMO
    g0, z0, o0 = nc, nc + nba, nc + 2 * nba

    p = pl.pallas_call(
        _p_body,
        out_shape=jax.ShapeDtypeStruct((n, d2), jnp.float32),
    )(features, W_base, W_mean, W_logstd)

    def adj_map(i):
        return (jnp.where(i < z0, jnp.clip(i - g0, 0, nba - 1),
                          jnp.where(i < o0, i - z0, nba - 1)), 0)

    def p_map(i):
        return (jnp.clip(i, 0, nc - 1), 0)

    def noise_map(i):
        return (jnp.clip(i - z0, 0, nba - 1), 0)

    def out_map(i):
        return (jnp.where(i >= o0, i - o0, 0), 0)

    body = functools.partial(_body, nc=nc, nba=nba, d_emb=d_emb)

    out = pl.pallas_call(
        body,
        grid=(nc + 2 * nba + nbo,),
        in_specs=[
            pl.BlockSpec((_BMA, n), adj_map),
            pl.BlockSpec((_BMP, d2), p_map),
            pl.BlockSpec((_BMA, d_emb), noise_map),
        ],
        out_specs=pl.BlockSpec((_BMO, n), out_map),
        out_shape=jax.ShapeDtypeStruct((n, n), jnp.float32),
        scratch_shapes=[
            pltpu.VMEM((n, d2), jnp.float32),
            pltpu.VMEM((n, d2), jnp.float32),
            pltpu.VMEM((n, d_emb), jnp.float32),
        ],
        compiler_params=pltpu.CompilerParams(
            vmem_limit_bytes=100 * 1024 * 1024),
    )(adj, p, noise)

    return out


# speedup vs baseline: 1.0165x; 1.0149x over previous
import functools

import jax
import jax.numpy as jnp
from jax import lax
from jax.experimental import pallas as pl
from jax.experimental.pallas import tpu as pltpu

_BMA = 256
_BMO = 200


def _body(adj_ref, f_ref, wb_ref, wm_ref, wl_ref, noise_ref, o_ref,
          p_ref, g_ref, z_ref, *, n, nba, nbo, d_emb):
    i = pl.program_id(0)

    @pl.when(i == 0)
    def _phase_p():
        wcat = jnp.concatenate([wm_ref[...], wl_ref[...]], axis=1)
        wc = jnp.dot(wb_ref[...], wcat, preferred_element_type=jnp.float32)
        p_ref[...] = jnp.dot(f_ref[...], wc,
                             preferred_element_type=jnp.float32)

    @pl.when(i < nba)
    def _phase_g():
        r = i * _BMA
        g_ref[pl.ds(r, _BMA), :] = jnp.dot(
            adj_ref[...], p_ref[...],
            preferred_element_type=jnp.float32)

    @pl.when((i >= nba) & (i < 2 * nba))
    def _phase_z():
        r = (i - nba) * _BMA
        m = jnp.maximum(jnp.dot(adj_ref[...], g_ref[:n, :],
                                preferred_element_type=jnp.float32), 0.0)
        mean = m[:, :d_emb]
        logstd = m[:, d_emb:]
        z_ref[pl.ds(r, _BMA), :] = (
            noise_ref[...] * jnp.exp(logstd) + mean)

    @pl.when(i >= 2 * nba)
    def _phase_out():
        r = (i - 2 * nba) * _BMO
        zi = z_ref[pl.ds(r, _BMO), :]
        o_ref[...] = lax.dot_general(
            zi, z_ref[:n, :], (((1,), (1,)), ((), ())),
            preferred_element_type=jnp.float32)


def kernel(adj, features, W_base, W_mean, W_logstd, noise):
    n, d_in = features.shape
    d_hid = W_base.shape[1]
    d_emb = W_mean.shape[1]
    d2 = 2 * d_emb
    nba = -(-n // _BMA)
    nbo = n // _BMO
    npad = nba * _BMA

    def adj_map(i):
        return (jnp.where(i < nba, i,
                          jnp.where(i < 2 * nba, i - nba, nba - 1)), 0)

    def noise_map(i):
        return (jnp.clip(i - nba, 0, nba - 1), 0)

    def out_map(i):
        return (jnp.where(i >= 2 * nba, i - 2 * nba, 0), 0)

    body = functools.partial(_body, n=n, nba=nba, nbo=nbo, d_emb=d_emb)

    out = pl.pallas_call(
        body,
        grid=(2 * nba + nbo,),
        in_specs=[
            pl.BlockSpec((_BMA, n), adj_map),
            pl.BlockSpec((n, d_in), lambda i: (0, 0)),
            pl.BlockSpec((d_in, d_hid), lambda i: (0, 0)),
            pl.BlockSpec((d_hid, d_emb), lambda i: (0, 0)),
            pl.BlockSpec((d_hid, d_emb), lambda i: (0, 0)),
            pl.BlockSpec((_BMA, d_emb), noise_map),
        ],
        out_specs=pl.BlockSpec((_BMO, n), out_map),
        out_shape=jax.ShapeDtypeStruct((n, n), jnp.float32),
        scratch_shapes=[
            pltpu.VMEM((n, d2), jnp.float32),
            pltpu.VMEM((npad, d2), jnp.float32),
            pltpu.VMEM((npad, d_emb), jnp.float32),
        ],
    )(adj, features, W_base, W_mean, W_logstd, noise)

    return out


# speedup vs baseline: 1.0169x; 1.0004x over previous
import functools

import jax
import jax.numpy as jnp
from jax import lax
from jax.experimental import pallas as pl
from jax.experimental.pallas import tpu as pltpu

_BMA = 320
_BMO = 200


def _body(adj_ref, f_ref, wb_ref, wm_ref, wl_ref, noise_ref, o_ref,
          p_ref, g_ref, z_ref, *, n, nba, nbo, d_emb):
    i = pl.program_id(0)

    @pl.when(i == 0)
    def _phase_p():
        wcat = jnp.concatenate([wm_ref[...], wl_ref[...]], axis=1)
        wc = jnp.dot(wb_ref[...], wcat, preferred_element_type=jnp.float32)
        p_ref[...] = jnp.dot(f_ref[...], wc,
                             preferred_element_type=jnp.float32)

    @pl.when(i < nba)
    def _phase_g():
        r = i * _BMA
        g_ref[pl.ds(r, _BMA), :] = jnp.dot(
            adj_ref[...], p_ref[...],
            preferred_element_type=jnp.float32)

    @pl.when((i >= nba) & (i < 2 * nba))
    def _phase_z():
        r = (i - nba) * _BMA
        m = jnp.maximum(jnp.dot(adj_ref[...], g_ref[:n, :],
                                preferred_element_type=jnp.float32), 0.0)
        mean = m[:, :d_emb]
        logstd = m[:, d_emb:]
        z_ref[pl.ds(r, _BMA), :] = (
            noise_ref[...] * jnp.exp(logstd) + mean)

    @pl.when(i >= 2 * nba)
    def _phase_out():
        r = (i - 2 * nba) * _BMO
        zi = z_ref[pl.ds(r, _BMO), :]
        o_ref[...] = lax.dot_general(
            zi, z_ref[:n, :], (((1,), (1,)), ((), ())),
            preferred_element_type=jnp.float32)


def kernel(adj, features, W_base, W_mean, W_logstd, noise):
    n, d_in = features.shape
    d_hid = W_base.shape[1]
    d_emb = W_mean.shape[1]
    d2 = 2 * d_emb
    nba = -(-n // _BMA)
    nbo = n // _BMO
    npad = nba * _BMA

    def adj_map(i):
        return (jnp.where(i < nba, i,
                          jnp.where(i < 2 * nba, i - nba, nba - 1)), 0)

    def noise_map(i):
        return (jnp.clip(i - nba, 0, nba - 1), 0)

    def out_map(i):
        return (jnp.where(i >= 2 * nba, i - 2 * nba, 0), 0)

    body = functools.partial(_body, n=n, nba=nba, nbo=nbo, d_emb=d_emb)

    out = pl.pallas_call(
        body,
        grid=(2 * nba + nbo,),
        in_specs=[
            pl.BlockSpec((_BMA, n), adj_map),
            pl.BlockSpec((n, d_in), lambda i: (0, 0)),
            pl.BlockSpec((d_in, d_hid), lambda i: (0, 0)),
            pl.BlockSpec((d_hid, d_emb), lambda i: (0, 0)),
            pl.BlockSpec((d_hid, d_emb), lambda i: (0, 0)),
            pl.BlockSpec((_BMA, d_emb), noise_map),
        ],
        out_specs=pl.BlockSpec((_BMO, n), out_map),
        out_shape=jax.ShapeDtypeStruct((n, n), jnp.float32),
        scratch_shapes=[
            pltpu.VMEM((n, d2), jnp.float32),
            pltpu.VMEM((npad, d2), jnp.float32),
            pltpu.VMEM((npad, d_emb), jnp.float32),
        ],
        compiler_params=pltpu.CompilerParams(
            vmem_limit_bytes=100 * 1024 * 1024),
    )(adj, features, W_base, W_mean, W_logstd, noise)

    return out


# speedup vs baseline: 1.0244x; 1.0074x over previous
import functools

import jax
import jax.numpy as jnp
from jax import lax
from jax.experimental import pallas as pl
from jax.experimental.pallas import tpu as pltpu

_BMA = 304
_BMO = 256


def _body(adj_ref, f_ref, wb_ref, wm_ref, wl_ref, noise_ref, o_ref,
          p_ref, g_ref, z_ref, *, n, nba, nbo, d_emb):
    i = pl.program_id(0)

    @pl.when(i == 0)
    def _phase_p():
        wcat = jnp.concatenate([wm_ref[...], wl_ref[...]], axis=1)
        wc = jnp.dot(wb_ref[...], wcat, preferred_element_type=jnp.float32)
        p_ref[...] = jnp.dot(f_ref[...], wc,
                             preferred_element_type=jnp.float32)

    @pl.when(i < nba)
    def _phase_g():
        r = i * _BMA
        g_ref[pl.ds(r, _BMA), :] = jnp.dot(
            adj_ref[...], p_ref[...],
            preferred_element_type=jnp.float32)

    @pl.when((i >= nba) & (i < 2 * nba))
    def _phase_z():
        r = (i - nba) * _BMA
        m = jnp.maximum(jnp.dot(adj_ref[...], g_ref[:n, :],
                                preferred_element_type=jnp.float32), 0.0)
        mean = m[:, :d_emb]
        logstd = m[:, d_emb:]
        z_ref[pl.ds(r, _BMA), :] = (
            noise_ref[...] * jnp.exp(logstd) + mean)

    @pl.when(i >= 2 * nba)
    def _phase_out():
        r = (i - 2 * nba) * _BMO
        zi = z_ref[pl.ds(r, _BMO), :]
        o_ref[...] = lax.dot_general(
            zi, z_ref[:n, :], (((1,), (1,)), ((), ())),
            preferred_element_type=jnp.float32)


def kernel(adj, features, W_base, W_mean, W_logstd, noise):
    n, d_in = features.shape
    d_hid = W_base.shape[1]
    d_emb = W_mean.shape[1]
    d2 = 2 * d_emb
    nba = -(-n // _BMA)
    nbo = -(-n // _BMO)
    npad = max(nba * _BMA, nbo * _BMO)

    def adj_map(i):
        return (jnp.where(i < nba, i,
                          jnp.where(i < 2 * nba, i - nba, nba - 1)), 0)

    def noise_map(i):
        return (jnp.clip(i - nba, 0, nba - 1), 0)

    def out_map(i):
        return (jnp.where(i >= 2 * nba, i - 2 * nba, 0), 0)

    body = functools.partial(_body, n=n, nba=nba, nbo=nbo, d_emb=d_emb)

    out = pl.pallas_call(
        body,
        grid=(2 * nba + nbo,),
        in_specs=[
            pl.BlockSpec((_BMA, n), adj_map),
            pl.BlockSpec((n, d_in), lambda i: (0, 0)),
            pl.BlockSpec((d_in, d_hid), lambda i: (0, 0)),
            pl.BlockSpec((d_hid, d_emb), lambda i: (0, 0)),
            pl.BlockSpec((d_hid, d_emb), lambda i: (0, 0)),
            pl.BlockSpec((_BMA, d_emb), noise_map),
        ],
        out_specs=pl.BlockSpec((_BMO, n), out_map),
        out_shape=jax.ShapeDtypeStruct((n, n), jnp.float32),
        scratch_shapes=[
            pltpu.VMEM((n, d2), jnp.float32),
            pltpu.VMEM((npad, d2), jnp.float32),
            pltpu.VMEM((npad, d_emb), jnp.float32),
        ],
        compiler_params=pltpu.CompilerParams(
            vmem_limit_bytes=100 * 1024 * 1024),
    )(adj, features, W_base, W_mean, W_logstd, noise)

    return out
